# baseline (speedup 1.0000x reference)
import jax
import jax.numpy as jnp
from jax.experimental import pallas as pl

_B, _N, _M = 16, 1024, 2048
_DIN, _DE, _HID, _T, _K, _D = 1, 4, 32, 12, 2, 10
_CIN = _DIN + _HID
_NB = _N * _B
_F32 = jnp.float32
_BF16 = jnp.bfloat16


def _dot(a, b):
    return jnp.dot(a.astype(_BF16), b.astype(_BF16),
                   preferred_element_type=_F32)


def _dot_nt(a, b):
    return jax.lax.dot_general(
        a.astype(_BF16), b.astype(_BF16),
        dimension_numbers=(((1,), (1,)), ((), ())),
        preferred_element_type=_F32)


def _prep_kernel(e_ref, h_ref, l1_ref, xe_ref, hed_ref,
                 xt_ref, xf_ref, x1_ref,
                 a_ref, g_ref, ax_ref, gx_ref, z_ref, hh_ref, xw_ref):
    e = e_ref[...]
    r = jax.lax.dot_general(e, e, (((1,), (1,)), ((), ())),
                            preferred_element_type=_F32)
    r = jnp.maximum(r, 0.0)
    m = jnp.max(r, axis=1, keepdims=True)
    ex = jnp.exp(r - m)
    a = (ex / jnp.sum(ex, axis=1, keepdims=True)).astype(_BF16)
    a_ref[...] = a

    h = h_ref[...].astype(_BF16)
    de = jnp.clip(jnp.sum(h, axis=0, keepdims=True, dtype=_F32), 1.0, None)
    dv = jnp.clip(jnp.sum(h, axis=1, keepdims=True, dtype=_F32), 1.0, None)
    hs = h * (1.0 / de).astype(_BF16)
    g = (_dot_nt(hs, h) / dv).astype(_BF16)
    g_ref[...] = g

    x1 = x1_ref[...]
    ax_ref[...] = _dot(a, x1).astype(_BF16)
    gx_ref[...] = _dot(g, x1).astype(_BF16)

    l1xe = _dot(l1_ref[...], xe_ref[...]).astype(_BF16)
    both = jnp.concatenate([l1xe, hed_ref[...]], axis=1)
    zh = _dot(h, both)
    z_ref[...] = zh[:, :_B * _DE].astype(_BF16)
    hh_ref[...] = (zh[:, _B * _DE:] / dv).astype(_BF16)

    xt = xt_ref[...]
    mt = jnp.max(xt, axis=1, keepdims=True)
    et2 = jnp.exp(xt - mt)
    ta = et2 / jnp.sum(et2, axis=1, keepdims=True)
    xw = jnp.sum(xf_ref[...] * ta[None, :, :], axis=-1)
    xw_ref[...] = xw.astype(_BF16)


def _gmm_kernel(a_ref, g_ref, x_ref, ax_ref, gx_ref):
    x = x_ref[...]
    ax_ref[...] = _dot(a_ref[...], x).astype(_BF16)
    gx_ref[...] = _dot(g_ref[...], x).astype(_BF16)


def _gcn_rows(x_r, ax_r, gx_r, s_r, wp66, whyp, whed, whnd, we, bp, wt2,
              dout):
    u66 = jnp.concatenate([x_r, ax_r], axis=1)
    y = _dot(u66, wp66)
    ww1 = _dot(whed, whyp)
    ww2 = _dot(whnd, whyp)
    w65 = jnp.concatenate([whyp, ww1, ww2, we, bp, wt2], axis=0)
    e_r = s_r[:, 12:22]
    exw = e_r * s_r[:, 22:23]
    cat = jnp.concatenate([gx_r, s_r[:, :22], exw], axis=1)
    acc = _dot(cat, w65)
    for d in range(_D):
        ed = e_r[:, d:d + 1].astype(_F32)
        acc = acc + ed * y[:, d * dout:(d + 1) * dout]
    return acc


def _pw_gate_kernel(x_ref, ax_ref, gx_ref, s_ref, st_ref, wp_ref,
                    whyp_ref, whed_ref, whnd_ref,
                    we_ref, bp_ref, wt2_ref, zs_ref, r_ref):
    acc = _gcn_rows(x_ref[...], ax_ref[...], gx_ref[...], s_ref[...],
                    wp_ref[...], whyp_ref[...], whed_ref[...], whnd_ref[...],
                    we_ref[...], bp_ref[...], wt2_ref[...], 2 * _HID)
    zr = jax.nn.sigmoid(acc)
    zs_ref[...] = (zr[:, :_HID] * st_ref[...]).astype(_BF16)
    r_ref[...] = zr[:, _HID:]


def _pw_update_kernel(x_ref, ax_ref, gx_ref, s_ref, st_ref, r_ref,
                      wp_ref, whyp_ref, whed_ref,
                      whnd_ref, we_ref, bp_ref, wt2_ref, h_ref):
    acc = _gcn_rows(x_ref[...], ax_ref[...], gx_ref[...], s_ref[...],
                    wp_ref[...], whyp_ref[...], whed_ref[...], whnd_ref[...],
                    we_ref[...], bp_ref[...], wt2_ref[...], _HID)
    hc = jnp.tanh(acc)
    r = r_ref[...]
    h_ref[...] = r * st_ref[...] + (1.0 - r) * hc


def _full_spec(shape):
    return pl.BlockSpec(shape, lambda i: (0,) * len(shape))


_RBLK = 2048


def _pw_call(kern, extra_in, weights, out_dtypes):
    row_lanes = [_CIN, _CIN, _CIN, 23, _HID] + extra_in
    in_specs = [pl.BlockSpec((_RBLK, l), lambda i: (i, 0)) for l in row_lanes]
    in_specs += [_full_spec(w.shape) for w in weights]
    out_specs = [pl.BlockSpec((_RBLK, _HID), lambda i: (i, 0))
                 for _ in out_dtypes]
    out_shape = [jax.ShapeDtypeStruct((_NB, _HID), dt) for dt in out_dtypes]
    return pl.pallas_call(
        kern,
        grid=(_NB // _RBLK,),
        in_specs=in_specs,
        out_specs=out_specs if len(out_specs) > 1 else out_specs[0],
        out_shape=out_shape if len(out_shape) > 1 else out_shape[0],
    )


def kernel(x, state, x_full, node_embeddings, x_time, x_e, hodge_laplacian,
           incidence_matrix, hyper_edge_data, hyper_node_data,
           gate_weights_pool, gate_bias_pool, gate_weights_T, gate_W_hed,
           gate_W_hnd, gate_W_hyper, gate_W_e,
           update_weights_pool, update_bias_pool, update_weights_T,
           update_W_hed, update_W_hnd, update_W_hyper, update_W_e):
    E = node_embeddings

    x_t = jnp.transpose(x, (1, 0, 2))
    state_t = jnp.transpose(state, (1, 0, 2))
    x1 = jnp.concatenate([x_t, state_t], axis=-1).astype(_BF16)
    x1_mat = x1.reshape(_N, _B * _CIN)
    x1_rows = x1.reshape(_NB, _CIN)
    state_rows = state_t.reshape(_NB, _HID)
    xe_t = jnp.transpose(x_e, (1, 0, 2)).reshape(_M, _B * _DE).astype(_BF16)
    hed_t = (jnp.transpose(hyper_edge_data, (1, 0, 2))
             .reshape(_M, _B * _DE).astype(_BF16))
    hnd_rows = (jnp.transpose(hyper_node_data, (1, 0, 2))
                .reshape(_NB, _DE).astype(_BF16))
    xfull3 = jnp.transpose(x_full[..., 0], (2, 0, 1))
    e_rows = (jnp.broadcast_to(E[:, None, :], (_N, _B, _D))
              .reshape(_NB, _D).astype(_BF16))

    def prep_pool(pool):
        dout = pool.shape[-1]
        return (jnp.transpose(pool, (1, 2, 0, 3))
                .reshape(_K * _CIN, _D * dout).astype(_BF16))
    g_wp = prep_pool(gate_weights_pool)
    u_wp = prep_pool(update_weights_pool)
    g_wt2 = gate_weights_T.reshape(_D, 2 * _HID)
    u_wt2 = update_weights_T.reshape(_D, _HID)

    A, G, ax1, gx1, Z, Hh, xw = pl.pallas_call(
        _prep_kernel,
        out_shape=[jax.ShapeDtypeStruct((_N, _N), _BF16),
                   jax.ShapeDtypeStruct((_N, _N), _BF16),
                   jax.ShapeDtypeStruct((_N, _B * _CIN), _BF16),
                   jax.ShapeDtypeStruct((_N, _B * _CIN), _BF16),
                   jax.ShapeDtypeStruct((_N, _B * _DE), _BF16),
                   jax.ShapeDtypeStruct((_N, _B * _DE), _BF16),
                   jax.ShapeDtypeStruct((_N, _B), _BF16)],
    )(E, incidence_matrix, hodge_laplacian, xe_t, hed_t,
      x_time, xfull3, x1_mat)
    s23 = jnp.concatenate(
        [Hh.reshape(_NB, _DE), hnd_rows, Z.reshape(_NB, _DE), e_rows,
         xw.reshape(_NB, 1)], axis=1)

    zs_rows, r_rows = _pw_call(
        _pw_gate_kernel, [],
        [g_wp, gate_W_hyper, gate_W_hed, gate_W_hnd, gate_W_e,
         gate_bias_pool, g_wt2],
        [_BF16, _F32],
    )(x1_rows, ax1.reshape(_NB, _CIN), gx1.reshape(_NB, _CIN),
      s23, state_rows,
      g_wp, gate_W_hyper, gate_W_hed, gate_W_hnd, gate_W_e,
      gate_bias_pool, g_wt2)

    x2 = jnp.concatenate([x_t.astype(_BF16),
                          zs_rows.reshape(_N, _B, _HID)], axis=-1)
    ax2, gx2 = pl.pallas_call(
        _gmm_kernel,
        out_shape=[jax.ShapeDtypeStruct((_N, _B * _CIN), _BF16)] * 2,
    )(A, G, x2.reshape(_N, _B * _CIN))
    h_rows = _pw_call(
        _pw_update_kernel, [_HID],
        [u_wp, update_W_hyper, update_W_hed, update_W_hnd, update_W_e,
         update_bias_pool, u_wt2],
        [_F32],
    )(x2.reshape(_NB, _CIN), ax2.reshape(_NB, _CIN), gx2.reshape(_NB, _CIN),
      s23, state_rows, r_rows,
      u_wp, update_W_hyper, update_W_hed, update_W_hnd, update_W_e,
      update_bias_pool, u_wt2)

    return jnp.transpose(h_rows.reshape(_N, _B, _HID), (1, 0, 2))


# speedup vs baseline: 1.0575x; 1.0575x over previous
import jax
import jax.numpy as jnp
from jax.experimental import pallas as pl

_B, _N, _M = 16, 1024, 2048
_DIN, _DE, _HID, _T, _K, _D = 1, 4, 32, 12, 2, 10
_CIN = _DIN + _HID
_NB = _N * _B
_F32 = jnp.float32
_BF16 = jnp.bfloat16


def _dot(a, b):
    return jnp.dot(a.astype(_BF16), b.astype(_BF16),
                   preferred_element_type=_F32)


def _dot_nt(a, b):
    return jax.lax.dot_general(
        a.astype(_BF16), b.astype(_BF16),
        dimension_numbers=(((1,), (1,)), ((), ())),
        preferred_element_type=_F32)


def _prep_kernel(e_ref, h_ref, l1_ref, xe_ref, hed_ref,
                 xt_ref, xf_ref, x1_ref,
                 a_ref, g_ref, ax_ref, gx_ref, z_ref, hh_ref, xw_ref):
    e = e_ref[...]
    r = jax.lax.dot_general(e, e, (((1,), (1,)), ((), ())),
                            preferred_element_type=_F32)
    r = jnp.maximum(r, 0.0)
    m = jnp.max(r, axis=1, keepdims=True)
    ex = jnp.exp(r - m)
    a = (ex / jnp.sum(ex, axis=1, keepdims=True)).astype(_BF16)
    a_ref[...] = a

    h = h_ref[...].astype(_BF16)
    de = jnp.clip(jnp.sum(h, axis=0, keepdims=True, dtype=_F32), 1.0, None)
    dv = jnp.clip(jnp.sum(h, axis=1, keepdims=True, dtype=_F32), 1.0, None)
    hs = h * (1.0 / de).astype(_BF16)
    g = (_dot_nt(hs, h) / dv).astype(_BF16)
    g_ref[...] = g

    x1 = x1_ref[...]
    ax_ref[...] = _dot(a, x1).astype(_BF16)
    gx_ref[...] = _dot(g, x1).astype(_BF16)

    l1xe = _dot(l1_ref[...], xe_ref[...]).astype(_BF16)
    both = jnp.concatenate([l1xe, hed_ref[...]], axis=1)
    zh = _dot(h, both)
    z_ref[...] = zh[:, :_B * _DE].astype(_BF16)
    hh_ref[...] = (zh[:, _B * _DE:] / dv).astype(_BF16)

    xt = xt_ref[...]
    mt = jnp.max(xt, axis=1, keepdims=True)
    et2 = jnp.exp(xt - mt)
    ta = et2 / jnp.sum(et2, axis=1, keepdims=True)
    xw_ref[...] = jnp.sum(xf_ref[...] * ta[None, :, :], axis=-1)


def _gmm_kernel(a_ref, g_ref, x_ref, ax_ref, gx_ref):
    x = x_ref[...]
    ax_ref[...] = _dot(a_ref[...], x).astype(_BF16)
    gx_ref[...] = _dot(g_ref[...], x).astype(_BF16)


def _gcn_rows(x_r, ax_r, gx_r, hh_r, hnd_r, z_r, xw_r, e_r,
              wp66, whyp, whed, whnd, we, bp, wt2, dout):
    u66 = jnp.concatenate([x_r, ax_r], axis=1)
    y = _dot(u66, wp66)
    ww1 = _dot(whed, whyp)
    ww2 = _dot(whnd, whyp)
    w55 = jnp.concatenate([whyp, ww1, ww2, we, bp], axis=0)
    cat = jnp.concatenate([gx_r, hh_r, hnd_r, z_r, e_r], axis=1)
    acc = _dot(cat, w55)
    acc = acc + xw_r * _dot(e_r, wt2)
    for d in range(_D):
        ed = e_r[:, d:d + 1].astype(_F32)
        acc = acc + ed * y[:, d * dout:(d + 1) * dout]
    return acc


def _pw_gate_kernel(x_ref, ax_ref, gx_ref, hh_ref, hnd_ref, z_ref, xw_ref,
                    e_ref, st_ref, wp_ref, whyp_ref, whed_ref, whnd_ref,
                    we_ref, bp_ref, wt2_ref, zs_ref, r_ref):
    acc = _gcn_rows(x_ref[...], ax_ref[...], gx_ref[...], hh_ref[...],
                    hnd_ref[...], z_ref[...], xw_ref[...], e_ref[...],
                    wp_ref[...], whyp_ref[...], whed_ref[...], whnd_ref[...],
                    we_ref[...], bp_ref[...], wt2_ref[...], 2 * _HID)
    zr = jax.nn.sigmoid(acc)
    zs_ref[...] = (zr[:, :_HID] * st_ref[...]).astype(_BF16)
    r_ref[...] = zr[:, _HID:]


def _pw_update_kernel(x_ref, ax_ref, gx_ref, hh_ref, hnd_ref, z_ref, xw_ref,
                      e_ref, st_ref, r_ref, wp_ref, whyp_ref, whed_ref,
                      whnd_ref, we_ref, bp_ref, wt2_ref, h_ref):
    acc = _gcn_rows(x_ref[...], ax_ref[...], gx_ref[...], hh_ref[...],
                    hnd_ref[...], z_ref[...], xw_ref[...], e_ref[...],
                    wp_ref[...], whyp_ref[...], whed_ref[...], whnd_ref[...],
                    we_ref[...], bp_ref[...], wt2_ref[...], _HID)
    hc = jnp.tanh(acc)
    r = r_ref[...]
    h_ref[...] = r * st_ref[...] + (1.0 - r) * hc


def _full_spec(shape):
    return pl.BlockSpec(shape, lambda i: (0,) * len(shape))


_RBLK = 2048


def _pw_call(kern, extra_in, weights, out_dtypes):
    row_lanes = [_CIN, _CIN, _CIN, _DE, _DE, _DE, 1, _D, _HID] + extra_in
    in_specs = [pl.BlockSpec((_RBLK, l), lambda i: (i, 0)) for l in row_lanes]
    in_specs += [_full_spec(w.shape) for w in weights]
    out_specs = [pl.BlockSpec((_RBLK, _HID), lambda i: (i, 0))
                 for _ in out_dtypes]
    out_shape = [jax.ShapeDtypeStruct((_NB, _HID), dt) for dt in out_dtypes]
    return pl.pallas_call(
        kern,
        grid=(_NB // _RBLK,),
        in_specs=in_specs,
        out_specs=out_specs if len(out_specs) > 1 else out_specs[0],
        out_shape=out_shape if len(out_shape) > 1 else out_shape[0],
    )


def kernel(x, state, x_full, node_embeddings, x_time, x_e, hodge_laplacian,
           incidence_matrix, hyper_edge_data, hyper_node_data,
           gate_weights_pool, gate_bias_pool, gate_weights_T, gate_W_hed,
           gate_W_hnd, gate_W_hyper, gate_W_e,
           update_weights_pool, update_bias_pool, update_weights_T,
           update_W_hed, update_W_hnd, update_W_hyper, update_W_e):
    E = node_embeddings

    x_t = jnp.transpose(x, (1, 0, 2))
    state_t = jnp.transpose(state, (1, 0, 2))
    x1 = jnp.concatenate([x_t, state_t], axis=-1).astype(_BF16)
    x1_mat = x1.reshape(_N, _B * _CIN)
    x1_rows = x1.reshape(_NB, _CIN)
    state_rows = state_t.reshape(_NB, _HID)
    xe_t = jnp.transpose(x_e, (1, 0, 2)).reshape(_M, _B * _DE).astype(_BF16)
    hed_t = (jnp.transpose(hyper_edge_data, (1, 0, 2))
             .reshape(_M, _B * _DE).astype(_BF16))
    hnd_rows = (jnp.transpose(hyper_node_data, (1, 0, 2))
                .reshape(_NB, _DE).astype(_BF16))
    xfull3 = jnp.transpose(x_full[..., 0], (2, 0, 1))
    e_rows = (jnp.broadcast_to(E[:, None, :], (_N, _B, _D))
              .reshape(_NB, _D).astype(_BF16))

    def prep_pool(pool):
        dout = pool.shape[-1]
        return (jnp.transpose(pool, (1, 2, 0, 3))
                .reshape(_K * _CIN, _D * dout).astype(_BF16))
    g_wp = prep_pool(gate_weights_pool)
    u_wp = prep_pool(update_weights_pool)
    g_wt2 = gate_weights_T.reshape(_D, 2 * _HID)
    u_wt2 = update_weights_T.reshape(_D, _HID)

    A, G, ax1, gx1, Z, Hh, xw = pl.pallas_call(
        _prep_kernel,
        out_shape=[jax.ShapeDtypeStruct((_N, _N), _BF16),
                   jax.ShapeDtypeStruct((_N, _N), _BF16),
                   jax.ShapeDtypeStruct((_N, _B * _CIN), _BF16),
                   jax.ShapeDtypeStruct((_N, _B * _CIN), _BF16),
                   jax.ShapeDtypeStruct((_N, _B * _DE), _BF16),
                   jax.ShapeDtypeStruct((_N, _B * _DE), _BF16),
                   jax.ShapeDtypeStruct((_N, _B), _F32)],
    )(E, incidence_matrix, hodge_laplacian, xe_t, hed_t,
      x_time, xfull3, x1_mat)
    z_rows = Z.reshape(_NB, _DE)
    hh_rows = Hh.reshape(_NB, _DE)
    xw_rows = xw.reshape(_NB, 1)

    zs_rows, r_rows = _pw_call(
        _pw_gate_kernel, [],
        [g_wp, gate_W_hyper, gate_W_hed, gate_W_hnd, gate_W_e,
         gate_bias_pool, g_wt2],
        [_BF16, _F32],
    )(x1_rows, ax1.reshape(_NB, _CIN), gx1.reshape(_NB, _CIN),
      hh_rows, hnd_rows, z_rows, xw_rows, e_rows, state_rows,
      g_wp, gate_W_hyper, gate_W_hed, gate_W_hnd, gate_W_e,
      gate_bias_pool, g_wt2)

    x2 = jnp.concatenate([x_t.astype(_BF16),
                          zs_rows.reshape(_N, _B, _HID)], axis=-1)
    ax2, gx2 = pl.pallas_call(
        _gmm_kernel,
        out_shape=[jax.ShapeDtypeStruct((_N, _B * _CIN), _BF16)] * 2,
    )(A, G, x2.reshape(_N, _B * _CIN))
    h_rows = _pw_call(
        _pw_update_kernel, [_HID],
        [u_wp, update_W_hyper, update_W_hed, update_W_hnd, update_W_e,
         update_bias_pool, u_wt2],
        [_F32],
    )(x2.reshape(_NB, _CIN), ax2.reshape(_NB, _CIN), gx2.reshape(_NB, _CIN),
      hh_rows, hnd_rows, z_rows, xw_rows, e_rows, state_rows, r_rows,
      u_wp, update_W_hyper, update_W_hed, update_W_hnd, update_W_e,
      update_bias_pool, u_wt2)

    return jnp.transpose(h_rows.reshape(_N, _B, _HID), (1, 0, 2))
